# baseline (speedup 1.0000x reference)
import functools
import jax
import jax.numpy as jnp
from jax import lax
from jax.experimental import pallas as pl
from jax.experimental.pallas import tpu as pltpu
from jax.experimental.pallas import tpu_sc as plsc

_B = 16384
_D = 64
_NW = 32
_BPW = _B // _NW
_CHUNK = 128
_NCH = _BPW // _CHUNK


def _make_kernel():
    mesh = plsc.VectorSubcoreMesh(core_axis_name="c", subcore_axis_name="s")

    @functools.partial(
        pl.kernel,
        mesh=mesh,
        out_type=jax.ShapeDtypeStruct((_NW, 16), jnp.float32),
        compiler_params=pltpu.CompilerParams(use_tc_tiling_on_sc=False),
        scratch_types=[
            pltpu.VMEM((_NCH, _CHUNK), jnp.int32),
            pltpu.VMEM((_BPW, _D), jnp.float32),
            pltpu.VMEM((_BPW, _D), jnp.float32),
            pltpu.VMEM((16,), jnp.float32),
            pltpu.SemaphoreType.DMA,
        ],
    )
    def k(feat_hbm, idx_hbm, table_hbm, out_hbm, idx_v, feat_v, rows_v, acc_v, sem):
        wid = lax.axis_index("s") * 2 + lax.axis_index("c")
        base = wid * _BPW
        pltpu.sync_copy(idx_hbm.at[pl.ds(wid * _NCH, _NCH)], idx_v)
        copies = []
        for j in range(_NCH):
            copies.append(
                pltpu.async_copy(
                    table_hbm.at[idx_v.at[j]],
                    rows_v.at[pl.ds(j * _CHUNK, _CHUNK)],
                    sem,
                )
            )
        pltpu.sync_copy(feat_hbm.at[pl.ds(base, _BPW)], feat_v)
        for c in copies:
            c.wait()

        zero = jnp.zeros((16,), jnp.float32)

        def body(i, accs):
            a0, a1, a2, a3 = accs
            d0 = feat_v[i, pl.ds(0, 16)] - rows_v[i, pl.ds(0, 16)]
            d1 = feat_v[i, pl.ds(16, 16)] - rows_v[i, pl.ds(16, 16)]
            d2 = feat_v[i, pl.ds(32, 16)] - rows_v[i, pl.ds(32, 16)]
            d3 = feat_v[i, pl.ds(48, 16)] - rows_v[i, pl.ds(48, 16)]
            return (a0 + d0 * d0, a1 + d1 * d1, a2 + d2 * d2, a3 + d3 * d3)

        a0, a1, a2, a3 = lax.fori_loop(0, _BPW, body, (zero, zero, zero, zero))
        acc_v[...] = (a0 + a1) + (a2 + a3)
        pltpu.sync_copy(acc_v, out_hbm.at[wid])

    return k


_sc_kernel = _make_kernel()


def kernel(feat, labels, centers):
    idx = labels.astype(jnp.int32).reshape(_NW * _NCH, _CHUNK)
    partials = _sc_kernel(feat, idx, centers)
    return jnp.sum(partials) * (0.5 / _B)
